# Initial kernel scaffold; baseline (speedup 1.0000x reference)
import jax
import jax.numpy as jnp
from jax.experimental import pallas as pl


def kernel(x, params, edge_index):
    raise NotImplementedError("write your pallas kernel here")


# speedup vs baseline: 2.3642x; 2.3642x over previous
import functools

import jax
import jax.numpy as jnp
from jax import lax
from jax.experimental import pallas as pl
from jax.experimental.pallas import tpu as pltpu
from jax.experimental.pallas import tpu_sc as plsc

F32 = jnp.float32
EPS = 1e-5

NC = 2
NS = 16
CHUNK = 128

_HI = jax.lax.Precision.HIGHEST


def _matmul(a, b):
    return lax.dot_general(a, b, (((1,), (0,)), ((), ())),
                           precision=_HI, preferred_element_type=F32)


def _tc_pre(h, wl, split_cols):
    n, _ = h.shape
    dout = wl.shape[1]

    din = h.shape[1]
    grid = (n // _BLK,)
    if split_cols:
        d2 = dout // 2

        def body(h_ref, w_ref, y_ref):
            y = _matmul(h_ref[...], w_ref[...])
            y_ref[0] = y[:, :d2]
            y_ref[1] = y[:, d2:]

        y2 = pl.pallas_call(
            body,
            grid=grid,
            in_specs=[pl.BlockSpec((_BLK, din), lambda i: (i, 0)),
                      pl.BlockSpec((din, dout), lambda i: (0, 0))],
            out_specs=pl.BlockSpec((2, _BLK, d2), lambda i: (0, i, 0)),
            out_shape=jax.ShapeDtypeStruct((2, n, d2), F32),
        )(h, wl)
        return y2.reshape(2 * n, d2)

    def body(h_ref, w_ref, y_ref):
        y_ref[...] = _matmul(h_ref[...], w_ref[...])

    return pl.pallas_call(
        body,
        grid=grid,
        in_specs=[pl.BlockSpec((_BLK, din), lambda i: (i, 0)),
                  pl.BlockSpec((din, dout), lambda i: (0, 0))],
        out_specs=pl.BlockSpec((_BLK, dout), lambda i: (i, 0)),
        out_shape=jax.ShapeDtypeStruct((n, dout), F32),
    )(h, wl)


def _tc_lin(h, wr, bl):
    n, _ = h.shape
    dout = wr.shape[1]

    din = h.shape[1]

    def body(h_ref, w_ref, b_ref, o_ref):
        o_ref[...] = _matmul(h_ref[...], w_ref[...]) + b_ref[...][None, :]

    return pl.pallas_call(
        body,
        grid=(n // _BLK,),
        in_specs=[pl.BlockSpec((_BLK, din), lambda i: (i, 0)),
                  pl.BlockSpec((din, dout), lambda i: (0, 0)),
                  pl.BlockSpec((dout,), lambda i: (0,))],
        out_specs=pl.BlockSpec((_BLK, dout), lambda i: (i, 0)),
        out_shape=jax.ShapeDtypeStruct((n, dout), F32),
    )(h, wr, bl)


def _tc_inv_cnt(cnt0, cnt1):
    n = cnt0.shape[0]

    def body(c0_ref, c1_ref, o_ref):
        cnt = c0_ref[...][:, :1] + c1_ref[...][:, :1]
        o_ref[...] = 1.0 / jnp.maximum(cnt, 1.0)

    return pl.pallas_call(
        body, out_shape=jax.ShapeDtypeStruct((n, 1), F32),
    )(cnt0, cnt1)


_BLK = 2000


def _tc_post_a(agg0, agg1, hw, inv_cnt, split_cols):
    n, dout = hw.shape
    d2 = agg0.shape[1]

    def body(a0_ref, a1_ref, hw_ref, ic_ref, z_ref, st_ref):
        i = pl.program_id(0)
        if split_cols:
            agg = jnp.concatenate([a0_ref[...], a1_ref[...]], axis=1)
        else:
            agg = a0_ref[...] + a1_ref[...]
        z = agg * ic_ref[...] + hw_ref[...]
        nrm = jnp.sqrt(jnp.sum(z * z, axis=1, keepdims=True))
        z = z / jnp.maximum(nrm, 1e-12)
        z_ref[...] = z
        s1 = jnp.sum(z, axis=0, keepdims=True)
        s2 = jnp.sum(z * z, axis=0, keepdims=True)
        st = jnp.concatenate([s1, s2, jnp.zeros((6, dout), F32)], axis=0)

        @pl.when(i == 0)
        def _():
            st_ref[...] = st

        @pl.when(i > 0)
        def _():
            st_ref[...] += st

    grid = (n // _BLK,)
    return pl.pallas_call(
        body,
        grid=grid,
        in_specs=[
            pl.BlockSpec((_BLK, d2), lambda i: (i, 0)),
            pl.BlockSpec((_BLK, d2), lambda i: (i, 0)),
            pl.BlockSpec((_BLK, dout), lambda i: (i, 0)),
            pl.BlockSpec((_BLK, 1), lambda i: (i, 0)),
        ],
        out_specs=(pl.BlockSpec((_BLK, dout), lambda i: (i, 0)),
                   pl.BlockSpec((8, dout), lambda i: (0, 0))),
        out_shape=(jax.ShapeDtypeStruct((n, dout), F32),
                   jax.ShapeDtypeStruct((8, dout), F32)),
    )(agg0, agg1, hw, inv_cnt)


def _tc_post_b(z, stats, gamma, beta, relu):
    n, dout = z.shape

    def body(z_ref, st_ref, g_ref, b_ref, o_ref):
        mu = st_ref[0:1, :] * (1.0 / n)
        var = st_ref[1:2, :] * (1.0 / n) - mu * mu
        scale = g_ref[...][None, :] * lax.rsqrt(var + EPS)
        o = (z_ref[...] - mu) * scale + b_ref[...][None, :]
        if relu:
            o = jnp.maximum(o, 0.0)
        o_ref[...] = o

    grid = (n // _BLK,)
    return pl.pallas_call(
        body,
        grid=grid,
        in_specs=[
            pl.BlockSpec((_BLK, dout), lambda i: (i, 0)),
            pl.BlockSpec((8, dout), lambda i: (0, 0)),
            pl.BlockSpec((dout,), lambda i: (0,)),
            pl.BlockSpec((dout,), lambda i: (0,)),
        ],
        out_specs=pl.BlockSpec((_BLK, dout), lambda i: (i, 0)),
        out_shape=jax.ShapeDtypeStruct((n, dout), F32),
    )(z, stats, gamma, beta)


def _tc_heads(h, w1_all, b1_all, w2_all, b2_all, n_heads, dh):
    n = h.shape[0]
    blk = 1000

    def body(h_ref, w1_ref, b1_ref, w2_ref, b2_ref, o_ref):
        t = _matmul(h_ref[...], w1_ref[...]) + b1_ref[...][None, :]
        t = jnp.maximum(t, 0.0) * w2_ref[...][None, :]
        cols = [jnp.sum(t[:, i * dh:(i + 1) * dh], axis=1, keepdims=True)
                for i in range(n_heads)]
        o_ref[...] = jnp.concatenate(cols, axis=1) + b2_ref[...][None, :]

    grid = (n // blk,)
    return pl.pallas_call(
        body,
        grid=grid,
        in_specs=[
            pl.BlockSpec((blk, h.shape[1]), lambda i: (i, 0)),
            pl.BlockSpec(w1_all.shape, lambda i: (0, 0)),
            pl.BlockSpec(b1_all.shape, lambda i: (0,)),
            pl.BlockSpec(w2_all.shape, lambda i: (0,)),
            pl.BlockSpec(b2_all.shape, lambda i: (0,)),
        ],
        out_specs=pl.BlockSpec((blk, n_heads), lambda i: (i, 0)),
        out_shape=jax.ShapeDtypeStruct((n, n_heads), F32),
    )(h, w1_all, b1_all, w2_all, b2_all)


def _sc_aggregate(y_flat, src_p, dst_p, zeros_rows, n, n_acc, e_pad,
                  split_cols):
    d2 = y_flat.shape[1]
    cores_per_sweep = 1 if split_cols else NC
    per_tile = e_pad // (NS * cores_per_sweep)
    chunks = per_tile // CHUNK
    zr = n_acc // NS
    mesh = plsc.VectorSubcoreMesh(core_axis_name="c", subcore_axis_name="s")

    @functools.partial(
        pl.kernel,
        out_type=jax.ShapeDtypeStruct((2, n_acc, d2), F32),
        mesh=mesh,
        scratch_types=[
            pltpu.VMEM((CHUNK,), jnp.int32),
            pltpu.VMEM((CHUNK,), jnp.int32),
            pltpu.VMEM((CHUNK, d2), F32),
            pltpu.VMEM_SHARED((n_acc, d2), F32),
        ],
    )
    def k(y_hbm, src_hbm, dst_hbm, z_hbm, o_hbm, sidx, didx, rows, acc):
        c = lax.axis_index("c")
        s = lax.axis_index("s")
        row0 = pl.multiple_of(s * zr, 8)

        pltpu.sync_copy(z_hbm, acc.at[pl.ds(row0, zr)])
        plsc.subcore_barrier()

        if split_cols:
            base = s * per_tile
            idx_off = c * n
        else:
            base = (c * NS + s) * per_tile
            idx_off = c * 0

        @pl.loop(0, chunks)
        def _(kk):
            off = base + kk * CHUNK
            pltpu.sync_copy(src_hbm.at[pl.ds(off, CHUNK)], sidx)
            pltpu.sync_copy(dst_hbm.at[pl.ds(off, CHUNK)], didx)
            if split_cols:
                for j in range(CHUNK // 16):
                    sl = pl.ds(j * 16, 16)
                    sidx[sl] = sidx[sl] + idx_off
            pltpu.sync_copy(y_hbm.at[sidx], rows)
            pltpu.sync_copy(rows, acc.at[didx], add=True)

        plsc.subcore_barrier()
        pltpu.sync_copy(acc.at[pl.ds(row0, zr)],
                        o_hbm.at[c, pl.ds(row0, zr)])

    return k(y_flat, src_p, dst_p, zeros_rows)


def _sc_count(dst_p, ones_rows, zeros_rows, n, n_acc, e_pad):
    w = ones_rows.shape[1]
    per_worker = e_pad // (NC * NS)
    chunks = per_worker // CHUNK
    zr = n_acc // NS
    mesh = plsc.VectorSubcoreMesh(core_axis_name="c", subcore_axis_name="s")

    @functools.partial(
        pl.kernel,
        out_type=jax.ShapeDtypeStruct((2, n_acc, w), F32),
        mesh=mesh,
        scratch_types=[
            pltpu.VMEM((CHUNK,), jnp.int32),
            pltpu.VMEM((CHUNK, w), F32),
            pltpu.VMEM_SHARED((n_acc, w), F32),
        ],
    )
    def k(dst_hbm, ones_hbm, z_hbm, o_hbm, didx, ones_v, acc):
        c = lax.axis_index("c")
        s = lax.axis_index("s")
        row0 = pl.multiple_of(s * zr, 8)

        pltpu.sync_copy(z_hbm, acc.at[pl.ds(row0, zr)])
        pltpu.sync_copy(ones_hbm, ones_v)
        plsc.subcore_barrier()

        base = (c * NS + s) * per_worker

        @pl.loop(0, chunks)
        def _(kk):
            off = base + kk * CHUNK
            pltpu.sync_copy(dst_hbm.at[pl.ds(off, CHUNK)], didx)
            pltpu.sync_copy(ones_v, acc.at[didx], add=True)

        plsc.subcore_barrier()
        pltpu.sync_copy(acc.at[pl.ds(row0, zr)],
                        o_hbm.at[c, pl.ds(row0, zr)])

    return k(dst_p, ones_rows, zeros_rows)


def kernel(x, params, edge_index):
    n = x.shape[0]
    e = edge_index.shape[1]

    e_pad = ((e + NS * CHUNK - 1) // (NS * CHUNK)) * (NS * CHUNK)
    if e_pad // (NC * NS) % CHUNK != 0:
        e_pad = ((e + NC * NS * CHUNK - 1) // (NC * NS * CHUNK)) * (NC * NS * CHUNK)
    pad = e_pad - e
    src_p = jnp.concatenate(
        [edge_index[0].astype(jnp.int32), jnp.zeros((pad,), jnp.int32)])
    dst_p = jnp.concatenate(
        [edge_index[1].astype(jnp.int32), jnp.full((pad,), n, jnp.int32)])

    n_acc = ((n + 1 + 8 * NS - 1) // (8 * NS)) * (8 * NS)
    zeros_rows = jnp.zeros((n_acc // NS, 128), F32)
    ones_rows = jnp.ones((CHUNK, 128), F32)

    cnt = _sc_count(dst_p, ones_rows, zeros_rows, n, n_acc, e_pad)
    inv_cnt = _tc_inv_cnt(cnt[0, :n], cnt[1, :n])

    h = x
    convs = [(params["conv1"], params["bn1"], True),
             (params["conv2"], params["bn2"], True),
             (params["conv3"], params["bn3"], True),
             (params["conv4"], params["bn4"], False)]
    for cp, bp, relu in convs:
        split_cols = cp["Wl"].shape[1] > 128
        y_flat = _tc_pre(h, cp["Wl"], split_cols)
        agg = _sc_aggregate(y_flat, src_p, dst_p,
                            zeros_rows[:, :y_flat.shape[1]],
                            n, n_acc, e_pad, split_cols)
        agg0, agg1 = agg[0, :n], agg[1, :n]
        hw = _tc_lin(h, cp["Wr"], cp["bl"])
        z, stats = _tc_post_a(agg0, agg1, hw, inv_cnt, split_cols)
        h = _tc_post_b(z, stats, bp["gamma"], bp["beta"], relu)

    heads = params["heads"]
    n_heads = len(heads)
    dh = heads[0]["W1"].shape[1]
    w1_all = jnp.concatenate([hp["W1"] for hp in heads], axis=1)
    b1_all = jnp.concatenate([hp["b1"] for hp in heads])
    w2_all = jnp.concatenate([hp["W2"][:, 0] for hp in heads])
    b2_all = jnp.stack([hp["b2"][0] for hp in heads])
    return _tc_heads(h, w1_all, b1_all, w2_all, b2_all, n_heads, dh)


# speedup vs baseline: 3.0335x; 1.2831x over previous
import functools

import jax
import jax.numpy as jnp
from jax import lax
from jax.experimental import pallas as pl
from jax.experimental.pallas import tpu as pltpu
from jax.experimental.pallas import tpu_sc as plsc

F32 = jnp.float32
EPS = 1e-5

NC = 2
NS = 16
CHUNK = 128

_HI = jax.lax.Precision.HIGHEST


def _matmul(a, b):
    return lax.dot_general(a, b, (((1,), (0,)), ((), ())),
                           precision=_HI, preferred_element_type=F32)


def _tc_pre(h, wl, split_cols):
    n, _ = h.shape
    dout = wl.shape[1]

    din = h.shape[1]
    grid = (n // _BLK,)
    if split_cols:
        d2 = dout // 2

        def body(h_ref, w_ref, y_ref):
            y = _matmul(h_ref[...], w_ref[...])
            y_ref[0] = y[:, :d2]
            y_ref[1] = y[:, d2:]

        y2 = pl.pallas_call(
            body,
            grid=grid,
            in_specs=[pl.BlockSpec((_BLK, din), lambda i: (i, 0)),
                      pl.BlockSpec((din, dout), lambda i: (0, 0))],
            out_specs=pl.BlockSpec((2, _BLK, d2), lambda i: (0, i, 0)),
            out_shape=jax.ShapeDtypeStruct((2, n, d2), F32),
        )(h, wl)
        return y2.reshape(2 * n, d2)

    def body(h_ref, w_ref, y_ref):
        y_ref[...] = _matmul(h_ref[...], w_ref[...])

    return pl.pallas_call(
        body,
        grid=grid,
        in_specs=[pl.BlockSpec((_BLK, din), lambda i: (i, 0)),
                  pl.BlockSpec((din, dout), lambda i: (0, 0))],
        out_specs=pl.BlockSpec((_BLK, dout), lambda i: (i, 0)),
        out_shape=jax.ShapeDtypeStruct((n, dout), F32),
    )(h, wl)


def _tc_lin(h, wr, bl):
    n, _ = h.shape
    dout = wr.shape[1]

    din = h.shape[1]

    def body(h_ref, w_ref, b_ref, o_ref):
        o_ref[...] = _matmul(h_ref[...], w_ref[...]) + b_ref[...][None, :]

    return pl.pallas_call(
        body,
        grid=(n // _BLK,),
        in_specs=[pl.BlockSpec((_BLK, din), lambda i: (i, 0)),
                  pl.BlockSpec((din, dout), lambda i: (0, 0)),
                  pl.BlockSpec((dout,), lambda i: (0,))],
        out_specs=pl.BlockSpec((_BLK, dout), lambda i: (i, 0)),
        out_shape=jax.ShapeDtypeStruct((n, dout), F32),
    )(h, wr, bl)


def _tc_inv_cnt(cnt0, cnt1):
    n = cnt0.shape[0]

    def body(c0_ref, c1_ref, o_ref):
        cnt = c0_ref[...][:, :1] + c1_ref[...][:, :1]
        o_ref[...] = 1.0 / jnp.maximum(cnt, 1.0)

    return pl.pallas_call(
        body, out_shape=jax.ShapeDtypeStruct((n, 1), F32),
    )(cnt0, cnt1)


_BLK = 2000


def _tc_post_a(agg0, agg1, hw, inv_cnt, split_cols):
    n, dout = hw.shape
    d2 = agg0.shape[1]

    def body(a0_ref, a1_ref, hw_ref, ic_ref, z_ref, st_ref):
        i = pl.program_id(0)
        if split_cols:
            agg = jnp.concatenate([a0_ref[...], a1_ref[...]], axis=1)
        else:
            agg = a0_ref[...] + a1_ref[...]
        z = agg * ic_ref[...] + hw_ref[...]
        nrm = jnp.sqrt(jnp.sum(z * z, axis=1, keepdims=True))
        z = z / jnp.maximum(nrm, 1e-12)
        z_ref[...] = z
        s1 = jnp.sum(z, axis=0, keepdims=True)
        s2 = jnp.sum(z * z, axis=0, keepdims=True)
        st = jnp.concatenate([s1, s2, jnp.zeros((6, dout), F32)], axis=0)

        @pl.when(i == 0)
        def _():
            st_ref[...] = st

        @pl.when(i > 0)
        def _():
            st_ref[...] += st

    grid = (n // _BLK,)
    return pl.pallas_call(
        body,
        grid=grid,
        in_specs=[
            pl.BlockSpec((_BLK, d2), lambda i: (i, 0)),
            pl.BlockSpec((_BLK, d2), lambda i: (i, 0)),
            pl.BlockSpec((_BLK, dout), lambda i: (i, 0)),
            pl.BlockSpec((_BLK, 1), lambda i: (i, 0)),
        ],
        out_specs=(pl.BlockSpec((_BLK, dout), lambda i: (i, 0)),
                   pl.BlockSpec((8, dout), lambda i: (0, 0))),
        out_shape=(jax.ShapeDtypeStruct((n, dout), F32),
                   jax.ShapeDtypeStruct((8, dout), F32)),
    )(agg0, agg1, hw, inv_cnt)


def _tc_post_b(z, stats, gamma, beta, relu):
    n, dout = z.shape

    def body(z_ref, st_ref, g_ref, b_ref, o_ref):
        mu = st_ref[0:1, :] * (1.0 / n)
        var = st_ref[1:2, :] * (1.0 / n) - mu * mu
        scale = g_ref[...][None, :] * lax.rsqrt(var + EPS)
        o = (z_ref[...] - mu) * scale + b_ref[...][None, :]
        if relu:
            o = jnp.maximum(o, 0.0)
        o_ref[...] = o

    grid = (n // _BLK,)
    return pl.pallas_call(
        body,
        grid=grid,
        in_specs=[
            pl.BlockSpec((_BLK, dout), lambda i: (i, 0)),
            pl.BlockSpec((8, dout), lambda i: (0, 0)),
            pl.BlockSpec((dout,), lambda i: (0,)),
            pl.BlockSpec((dout,), lambda i: (0,)),
        ],
        out_specs=pl.BlockSpec((_BLK, dout), lambda i: (i, 0)),
        out_shape=jax.ShapeDtypeStruct((n, dout), F32),
    )(z, stats, gamma, beta)


def _tc_heads(h, w1_all, b1_all, w2_all, b2_all, n_heads, dh):
    n = h.shape[0]
    blk = 1000

    def body(h_ref, w1_ref, b1_ref, w2_ref, b2_ref, o_ref):
        t = _matmul(h_ref[...], w1_ref[...]) + b1_ref[...][None, :]
        t = jnp.maximum(t, 0.0) * w2_ref[...][None, :]
        cols = [jnp.sum(t[:, i * dh:(i + 1) * dh], axis=1, keepdims=True)
                for i in range(n_heads)]
        o_ref[...] = jnp.concatenate(cols, axis=1) + b2_ref[...][None, :]

    grid = (n // blk,)
    return pl.pallas_call(
        body,
        grid=grid,
        in_specs=[
            pl.BlockSpec((blk, h.shape[1]), lambda i: (i, 0)),
            pl.BlockSpec(w1_all.shape, lambda i: (0, 0)),
            pl.BlockSpec(b1_all.shape, lambda i: (0,)),
            pl.BlockSpec(w2_all.shape, lambda i: (0,)),
            pl.BlockSpec(b2_all.shape, lambda i: (0,)),
        ],
        out_specs=pl.BlockSpec((blk, n_heads), lambda i: (i, 0)),
        out_shape=jax.ShapeDtypeStruct((n, n_heads), F32),
    )(h, w1_all, b1_all, w2_all, b2_all)


def _sc_aggregate(y_flat, src2, dst2, zeros_rows, n, n_acc, e_pad,
                  split_cols):
    d2 = y_flat.shape[1]
    nch = e_pad // CHUNK
    cores_per_sweep = 1 if split_cols else NC
    chunks = nch // (NS * cores_per_sweep)
    n_halves = 2 if split_cols else 1
    zr = n_acc // NS
    mesh = plsc.VectorSubcoreMesh(core_axis_name="c", subcore_axis_name="s")

    @functools.partial(
        pl.kernel,
        out_type=jax.ShapeDtypeStruct((2, n_acc, d2), F32),
        mesh=mesh,
        scratch_types=[
            pltpu.VMEM((chunks // n_halves, CHUNK), jnp.int32),
            pltpu.VMEM((chunks // n_halves, CHUNK), jnp.int32),
            pltpu.VMEM((CHUNK, d2), F32),
            pltpu.VMEM((CHUNK, d2), F32),
            pltpu.VMEM_SHARED((n_acc, d2), F32),
            pltpu.SemaphoreType.DMA,
            pltpu.SemaphoreType.DMA,
        ],
    )
    def k(y_hbm, src_hbm, dst_hbm, z_hbm, o_hbm,
          sidx, didx, rows0, rows1, acc, sem0, sem1):
        c = lax.axis_index("c")
        s = lax.axis_index("s")
        hh = chunks // n_halves
        row0 = pl.multiple_of(s * zr, 8)
        if split_cols:
            cb = s * chunks
            plane = c
        else:
            cb = (c * NS + s) * chunks
            plane = c * 0

        pltpu.sync_copy(z_hbm, acc.at[pl.ds(row0, zr)])
        plsc.subcore_barrier()

        def gather(kk, buf, sem):
            return pltpu.async_copy(y_hbm.at[sidx.at[kk]], buf, sem)

        for half in range(n_halves):
            pltpu.sync_copy(src_hbm.at[plane, pl.ds(cb + half * hh, hh)],
                            sidx)
            pltpu.sync_copy(dst_hbm.at[pl.ds(cb + half * hh, hh)], didx)

            gather(0, rows0, sem0)
            gather(1, rows1, sem1)

            @pl.loop(0, hh // 2)
            def _(p):
                k0 = p * 2
                pltpu.make_async_copy(
                    y_hbm.at[sidx.at[k0]], rows0, sem0).wait()
                pltpu.sync_copy(rows0, acc.at[didx.at[k0]], add=True)

                @pl.when(k0 + 2 < hh)
                def _():
                    gather(k0 + 2, rows0, sem0)

                pltpu.make_async_copy(
                    y_hbm.at[sidx.at[k0]], rows1, sem1).wait()
                pltpu.sync_copy(rows1, acc.at[didx.at[k0 + 1]], add=True)

                @pl.when(k0 + 3 < hh)
                def _():
                    gather(k0 + 3, rows1, sem1)

        plsc.subcore_barrier()
        pltpu.sync_copy(acc.at[pl.ds(row0, zr)],
                        o_hbm.at[c, pl.ds(row0, zr)])

    return k(y_flat, src2, dst2, zeros_rows)


def _sc_count(dst2, ones_rows, zeros_rows, n, n_acc, e_pad):
    w = ones_rows.shape[1]
    nch = e_pad // CHUNK
    chunks = nch // (NC * NS)
    zr = n_acc // NS
    mesh = plsc.VectorSubcoreMesh(core_axis_name="c", subcore_axis_name="s")

    @functools.partial(
        pl.kernel,
        out_type=jax.ShapeDtypeStruct((2, n_acc, w), F32),
        mesh=mesh,
        scratch_types=[
            pltpu.VMEM((chunks, CHUNK), jnp.int32),
            pltpu.VMEM((CHUNK, w), F32),
            pltpu.VMEM_SHARED((n_acc, w), F32),
        ],
    )
    def k(dst_hbm, ones_hbm, z_hbm, o_hbm, didx, ones_v, acc):
        c = lax.axis_index("c")
        s = lax.axis_index("s")
        row0 = pl.multiple_of(s * zr, 8)
        cb = (c * NS + s) * chunks

        pltpu.sync_copy(dst_hbm.at[pl.ds(cb, chunks)], didx)
        pltpu.sync_copy(z_hbm, acc.at[pl.ds(row0, zr)])
        pltpu.sync_copy(ones_hbm, ones_v)
        plsc.subcore_barrier()

        @pl.loop(0, chunks)
        def _(kk):
            pltpu.sync_copy(ones_v, acc.at[didx.at[kk]], add=True)

        plsc.subcore_barrier()
        pltpu.sync_copy(acc.at[pl.ds(row0, zr)],
                        o_hbm.at[c, pl.ds(row0, zr)])

    return k(dst2, ones_rows, zeros_rows)


def kernel(x, params, edge_index):
    n = x.shape[0]
    e = edge_index.shape[1]

    e_pad = ((e + NS * CHUNK - 1) // (NS * CHUNK)) * (NS * CHUNK)
    if e_pad // (NC * NS) % CHUNK != 0:
        e_pad = ((e + NC * NS * CHUNK - 1) // (NC * NS * CHUNK)) * (NC * NS * CHUNK)
    pad = e_pad - e
    nch = e_pad // CHUNK
    src_p = jnp.concatenate(
        [edge_index[0].astype(jnp.int32), jnp.zeros((pad,), jnp.int32)])
    dst_p = jnp.concatenate(
        [edge_index[1].astype(jnp.int32), jnp.full((pad,), n, jnp.int32)])
    src2 = jnp.stack([src_p, src_p + n]).reshape(2, nch, CHUNK)
    dst2 = dst_p.reshape(nch, CHUNK)

    n_acc = ((n + 1 + 8 * NS - 1) // (8 * NS)) * (8 * NS)
    zeros_rows = jnp.zeros((n_acc // NS, 128), F32)
    ones_rows = jnp.ones((CHUNK, 128), F32)

    cnt = _sc_count(dst2, ones_rows, zeros_rows, n, n_acc, e_pad)
    inv_cnt = _tc_inv_cnt(cnt[0, :n], cnt[1, :n])

    h = x
    convs = [(params["conv1"], params["bn1"], True),
             (params["conv2"], params["bn2"], True),
             (params["conv3"], params["bn3"], True),
             (params["conv4"], params["bn4"], False)]
    for cp, bp, relu in convs:
        split_cols = cp["Wl"].shape[1] > 128
        y_flat = _tc_pre(h, cp["Wl"], split_cols)
        agg = _sc_aggregate(y_flat, src2, dst2,
                            zeros_rows[:, :y_flat.shape[1]],
                            n, n_acc, e_pad, split_cols)
        agg0, agg1 = agg[0, :n], agg[1, :n]
        hw = _tc_lin(h, cp["Wr"], cp["bl"])
        z, stats = _tc_post_a(agg0, agg1, hw, inv_cnt, split_cols)
        h = _tc_post_b(z, stats, bp["gamma"], bp["beta"], relu)

    heads = params["heads"]
    n_heads = len(heads)
    dh = heads[0]["W1"].shape[1]
    w1_all = jnp.concatenate([hp["W1"] for hp in heads], axis=1)
    b1_all = jnp.concatenate([hp["b1"] for hp in heads])
    w2_all = jnp.concatenate([hp["W2"][:, 0] for hp in heads])
    b2_all = jnp.stack([hp["b2"][0] for hp in heads])
    return _tc_heads(h, w1_all, b1_all, w2_all, b2_all, n_heads, dh)
